# Initial kernel scaffold; baseline (speedup 1.0000x reference)
import jax
import jax.numpy as jnp
from jax.experimental import pallas as pl


def kernel(triple_ids, adj_rel, adj_nbr, rel_semantics, W_proj, b_proj, Wq, bq, Wk, bk, Wv, bv, Wo, bo, ln1_s, ln1_b, W1, b1, W2, b2, ln2_s, ln2_b):
    raise NotImplementedError("write your pallas kernel here")


# speedup vs baseline: 4.4018x; 4.4018x over previous
import functools

import jax
import jax.numpy as jnp
from jax import lax
from jax.experimental import pallas as pl
from jax.experimental.pallas import tpu as pltpu
from jax.experimental.pallas import tpu_sc as plsc

B = 32
DEG = 32
E = 2 * DEG
T = B * E
N_REL = 2000
NP = 2048
REL_DIM = 384
DIM = 256
TOPK = 8
LAYERS = 2
EPS = 1e-5
NC, NS = 2, 16
GROUP = 16


def _k1_body(rs_ref, wp_ref, bp_ref, proj_ref, g_ref):
    p = jnp.dot(rs_ref[...], wp_ref[...],
                preferred_element_type=jnp.float32) + bp_ref[...]
    proj_ref[...] = p
    norm = jnp.sqrt(jnp.sum(p * p, axis=-1, keepdims=True))
    nhat = p / jnp.maximum(norm, 1e-12)
    g_ref[...] = lax.dot_general(nhat, nhat, (((1,), (1,)), ((), ())),
                                 preferred_element_type=jnp.float32)


def _run_k1(rs_pad, w_proj, b_proj):
    return pl.pallas_call(
        _k1_body,
        out_shape=(jax.ShapeDtypeStruct((NP, DIM), jnp.float32),
                   jax.ShapeDtypeStruct((NP, NP), jnp.float32)),
    )(rs_pad, w_proj, b_proj.reshape(1, DIM))


def _sc_body(ents_hbm, adj_rel_hbm, adj_nbr_hbm, g_hbm, proj_hbm,
             states0_hbm, ctx_hbm,
             ents_v, pidx, hrel, hnbr, nrel, relvec, grows, selrows,
             sel_v, ctxbuf, sem):
    b = lax.axis_index("s") * NC + lax.axis_index("c")
    iota16 = lax.iota(jnp.int32, 16)
    neg_inf = jnp.float32(-jnp.inf)

    pltpu.sync_copy(ents_hbm, ents_v)
    h_spl = plsc.load_gather(ents_v, [jnp.full((16,), b, jnp.int32)])
    t_spl = plsc.load_gather(ents_v, [jnp.full((16,), b + B, jnp.int32)])
    pidx[...] = jnp.where(iota16 == 0, h_spl, t_spl)
    pltpu.async_copy(adj_rel_hbm.at[pidx], hrel, sem).wait()
    pltpu.async_copy(adj_nbr_hbm.at[pidx], hnbr, sem).wait()

    for h in range(2):
        pltpu.async_copy(adj_rel_hbm.at[hnbr.at[h, pl.ds(0, DEG)]],
                         nrel.at[pl.ds(DEG * h, DEG)], sem).wait()
        pltpu.async_copy(proj_hbm.at[hrel.at[h, pl.ds(0, DEG)]],
                         relvec, sem).wait()
        pltpu.sync_copy(relvec, states0_hbm.at[pl.ds(b * E + DEG * h, DEG)])

    for g in range(E // GROUP):
        h, part = g // 2, g % 2
        idx = hrel.at[h, pl.ds(16 * part, 16)]
        pltpu.async_copy(g_hbm.at[idx], grows, sem).wait()

        def tok_body(j, _):
            e = GROUP * g + j
            nid0 = nrel[e, pl.ds(0, 16)]
            nid1 = nrel[e, pl.ds(16, 16)]
            row = jnp.full((16,), j, jnp.int32)
            s0 = plsc.load_gather(grows, [row, nid0])
            s1 = plsc.load_gather(grows, [row, nid1])
            for it in range(TOPK):
                m = jnp.max(jnp.maximum(s0, s1))
                eq0 = s0 == m
                eq1 = s1 == m
                c0 = plsc.all_reduce_population_count(eq0)
                f0 = plsc.all_reduce_ffs(eq0)
                f1 = plsc.all_reduce_ffs(eq1)
                use0 = c0 > 0
                cand0 = nid0.at[f0].get(mode="promise_in_bounds")
                cand1 = nid1.at[f1].get(mode="promise_in_bounds")
                sel_id = jnp.where(use0, cand0, cand1)
                plsc.store_scatter(sel_v,
                                   [jnp.full((16,), j * TOPK + it, jnp.int32)],
                                   sel_id, mask=iota16 == 0)
                s0 = jnp.where(use0 & (iota16 == f0), neg_inf, s0)
                s1 = jnp.where((~use0) & (iota16 == f1), neg_inf, s1)
            return 0

        lax.fori_loop(0, GROUP, tok_body, 0)

        pltpu.async_copy(proj_hbm.at[sel_v], selrows, sem).wait()

        def ctx_body(j, _):
            def chunk_body(c, _):
                acc = selrows[j * TOPK + 0, pl.ds(16 * c, 16)]
                for i in range(1, TOPK):
                    acc = acc + selrows[j * TOPK + i, pl.ds(16 * c, 16)]
                ctxbuf[j, pl.ds(16 * c, 16)] = acc * jnp.float32(1.0 / TOPK)
                return 0
            lax.fori_loop(0, DIM // 16, chunk_body, 0)
            return 0

        lax.fori_loop(0, GROUP, ctx_body, 0)
        pltpu.sync_copy(ctxbuf, ctx_hbm.at[pl.ds(b * E + GROUP * g, GROUP)])


def _run_k2(ents, adj_rel_p, adj_nbr_p, g_mat, proj):
    mesh = plsc.VectorSubcoreMesh(core_axis_name="c", subcore_axis_name="s",
                                  num_cores=NC, num_subcores=NS)
    fn = pl.kernel(
        _sc_body,
        out_type=(jax.ShapeDtypeStruct((T, DIM), jnp.float32),
                  jax.ShapeDtypeStruct((T, DIM), jnp.float32)),
        mesh=mesh,
        compiler_params=pltpu.CompilerParams(needs_layout_passes=False),
        scratch_types=[
            pltpu.VMEM((128,), jnp.int32),
            pltpu.VMEM((16,), jnp.int32),
            pltpu.VMEM((16, 128), jnp.int32),
            pltpu.VMEM((16, 128), jnp.int32),
            pltpu.VMEM((E, 128), jnp.int32),
            pltpu.VMEM((DEG, DIM), jnp.float32),
            pltpu.VMEM((GROUP, NP), jnp.float32),
            pltpu.VMEM((GROUP * TOPK, DIM), jnp.float32),
            pltpu.VMEM((GROUP * TOPK,), jnp.int32),
            pltpu.VMEM((GROUP, DIM), jnp.float32),
            pltpu.SemaphoreType.DMA,
        ],
    )
    return fn(ents, adj_rel_p, adj_nbr_p, g_mat, proj)


def _layernorm(x, s, b):
    m = jnp.mean(x, axis=-1, keepdims=True)
    v = jnp.mean((x - m) ** 2, axis=-1, keepdims=True)
    return (x - m) / jnp.sqrt(v + EPS) * s + b


def _k3_body(states_ref, ctx_ref, wv_ref, bv_ref, wo_ref, bo_ref,
             ln1s_ref, ln1b_ref, w1_ref, b1_ref, w2_ref, b2_ref,
             ln2s_ref, ln2b_ref, out_ref):
    x = states_ref[...]
    ctx = ctx_ref[...]
    for l in range(LAYERS):
        v = jnp.dot(ctx, wv_ref[l], preferred_element_type=jnp.float32) \
            + bv_ref[l]
        attn = jnp.dot(v, wo_ref[l], preferred_element_type=jnp.float32) \
            + bo_ref[l]
        x = _layernorm(x + attn, ln1s_ref[l], ln1b_ref[l])
        h = jnp.dot(x, w1_ref[l], preferred_element_type=jnp.float32) \
            + b1_ref[l]
        h = 0.5 * h * (1.0 + lax.erf(h * jnp.float32(0.7071067811865476)))
        ff = jnp.dot(h, w2_ref[l], preferred_element_type=jnp.float32) \
            + b2_ref[l]
        x = _layernorm(x + ff, ln2s_ref[l], ln2b_ref[l])
    out_ref[...] = x


def _run_k3(states0, ctx, Wv, bv, Wo, bo, ln1_s, ln1_b, W1, b1, W2, b2,
            ln2_s, ln2_b):
    return pl.pallas_call(
        _k3_body,
        out_shape=jax.ShapeDtypeStruct((T, DIM), jnp.float32),
    )(states0, ctx, Wv, bv.reshape(LAYERS, 1, DIM), Wo,
      bo.reshape(LAYERS, 1, DIM), ln1_s.reshape(LAYERS, 1, DIM),
      ln1_b.reshape(LAYERS, 1, DIM), W1, b1.reshape(LAYERS, 1, 4 * DIM),
      W2, b2.reshape(LAYERS, 1, DIM), ln2_s.reshape(LAYERS, 1, DIM),
      ln2_b.reshape(LAYERS, 1, DIM))


def kernel(triple_ids, adj_rel, adj_nbr, rel_semantics, W_proj, b_proj,
           Wq, bq, Wk, bk, Wv, bv, Wo, bo, ln1_s, ln1_b,
           W1, b1, W2, b2, ln2_s, ln2_b):
    del Wq, bq, Wk, bk
    rs_pad = jnp.zeros((NP, REL_DIM), jnp.float32).at[:N_REL].set(rel_semantics)
    proj, g_mat = _run_k1(rs_pad, W_proj, b_proj)

    head = triple_ids[:, 0].astype(jnp.int32)
    tail = triple_ids[:, 2].astype(jnp.int32)
    ents = jnp.zeros((128,), jnp.int32).at[0:B].set(head).at[B:2 * B].set(tail)
    adj_rel_p = jnp.zeros((adj_rel.shape[0], 128), jnp.int32)
    adj_rel_p = adj_rel_p.at[:, :DEG].set(adj_rel.astype(jnp.int32))
    adj_nbr_p = jnp.zeros((adj_nbr.shape[0], 128), jnp.int32)
    adj_nbr_p = adj_nbr_p.at[:, :DEG].set(adj_nbr.astype(jnp.int32))

    states0, ctx = _run_k2(ents, adj_rel_p, adj_nbr_p, g_mat, proj)
    out = _run_k3(states0, ctx, Wv, bv, Wo, bo, ln1_s, ln1_b,
                  W1, b1, W2, b2, ln2_s, ln2_b)
    states = out.reshape(B, E, DIM)
    mask = jnp.ones((B, E), jnp.float32)
    return states, mask


# speedup vs baseline: 5.3565x; 1.2169x over previous
import jax
import jax.numpy as jnp
from jax import lax
from jax.experimental import pallas as pl
from jax.experimental.pallas import tpu as pltpu
from jax.experimental.pallas import tpu_sc as plsc

B = 32
DEG = 32
E = 2 * DEG
T = B * E
N_REL = 2000
NP = 2048
N_ENT = 10000
REL_DIM = 384
DIM = 256
TOPK = 8
LAYERS = 2
EPS = 1e-5
NC, NS = 2, 16
GROUP = 8
NGRP = E // GROUP


def _k1_body(rs_ref, wp_ref, bp_ref, ar_ref, an_ref,
             proj_ref, g_ref, arp_ref, anp_ref):
    p = jnp.dot(rs_ref[...], wp_ref[...],
                preferred_element_type=jnp.float32) + bp_ref[...]
    p = jnp.concatenate([p, jnp.zeros((NP - N_REL, DIM), jnp.float32)], axis=0)
    proj_ref[...] = p
    norm = jnp.sqrt(jnp.sum(p * p, axis=-1, keepdims=True))
    nhat = p / jnp.maximum(norm, 1e-12)
    g_ref[...] = lax.dot_general(nhat, nhat, (((1,), (1,)), ((), ())),
                                 preferred_element_type=jnp.float32)
    zpad = jnp.zeros((N_ENT, 128 - DEG), jnp.int32)
    arp_ref[...] = jnp.concatenate([ar_ref[...], zpad], axis=1)
    anp_ref[...] = jnp.concatenate([an_ref[...], zpad], axis=1)


def _run_k1(rel_semantics, w_proj, b_proj, adj_rel, adj_nbr):
    return pl.pallas_call(
        _k1_body,
        out_shape=(jax.ShapeDtypeStruct((NP, DIM), jnp.float32),
                   jax.ShapeDtypeStruct((NP, NP), jnp.float32),
                   jax.ShapeDtypeStruct((N_ENT, 128), jnp.int32),
                   jax.ShapeDtypeStruct((N_ENT, 128), jnp.int32)),
    )(rel_semantics, w_proj, b_proj.reshape(1, DIM), adj_rel, adj_nbr)


def _sc_body(ents_hbm, adj_rel_hbm, adj_nbr_hbm, g_hbm, proj_hbm,
             states0_hbm, ctx_hbm,
             ents_v, pidx, hrel, hnbr, nrel, relvec, ga, gb, sims,
             sela, selb, sra, srb, ctxbuf,
             sem_m, sem_ga, sem_gb, sem_sa, sem_sb):
    b = lax.axis_index("s") * NC + lax.axis_index("c")
    iota16 = lax.iota(jnp.int32, 16)
    neg_inf = jnp.float32(-jnp.inf)
    grows = (ga, gb)
    sels = (sela, selb)
    selrows = (sra, srb)
    sem_g = (sem_ga, sem_gb)
    sem_s = (sem_sa, sem_sb)

    pltpu.sync_copy(ents_hbm, ents_v)
    h_spl = plsc.load_gather(ents_v, [jnp.full((16,), b, jnp.int32)])
    t_spl = plsc.load_gather(ents_v, [jnp.full((16,), b + B, jnp.int32)])
    pidx[...] = jnp.where(iota16 == 0, h_spl, t_spl)
    cp_r = pltpu.async_copy(adj_rel_hbm.at[pidx], hrel, sem_m)
    cp_n = pltpu.async_copy(adj_nbr_hbm.at[pidx], hnbr, sem_m)
    cp_r.wait()
    cp_n.wait()

    cps = []
    for h in range(2):
        cps.append(pltpu.async_copy(adj_rel_hbm.at[hnbr.at[h, pl.ds(0, DEG)]],
                                    nrel.at[pl.ds(DEG * h, DEG)], sem_m))
        cps.append(pltpu.async_copy(proj_hbm.at[hrel.at[h, pl.ds(0, DEG)]],
                                    relvec.at[pl.ds(DEG * h, DEG)], sem_m))
    pltpu.async_copy(g_hbm.at[hrel.at[0, pl.ds(0, GROUP)]], ga, sem_ga)
    for cp in cps:
        cp.wait()
    pltpu.sync_copy(relvec, states0_hbm.at[pl.ds(b * E, E)])

    def ctx_accum(src, g):
        def ctx_body(j, _):
            def chunk_body(c, _):
                acc = src[j * TOPK + 0, pl.ds(16 * c, 16)]
                for i in range(1, TOPK):
                    acc = acc + src[j * TOPK + i, pl.ds(16 * c, 16)]
                ctxbuf[j, pl.ds(16 * c, 16)] = acc * jnp.float32(1.0 / TOPK)
                return 0
            lax.fori_loop(0, DIM // 16, chunk_body, 0)
            return 0
        lax.fori_loop(0, GROUP, ctx_body, 0)
        pltpu.sync_copy(ctxbuf, ctx_hbm.at[pl.ds(b * E + GROUP * g, GROUP)])

    sel_cps = [None, None]
    for g in range(NGRP):
        cur = g % 2
        pltpu.make_async_copy(
            g_hbm.at[hrel.at[g // 4, pl.ds((g % 4) * GROUP, GROUP)]],
            grows[cur], sem_g[cur]).wait()
        if g + 1 < NGRP:
            g2 = g + 1
            pltpu.async_copy(
                g_hbm.at[hrel.at[g2 // 4, pl.ds((g2 % 4) * GROUP, GROUP)]],
                grows[g2 % 2], sem_g[g2 % 2])
        for j in range(GROUP):
            e = GROUP * g + j
            rowj = jnp.full((16,), j, jnp.int32)
            sims[j, pl.ds(0, 16)] = plsc.load_gather(
                grows[cur], [rowj, nrel[e, pl.ds(0, 16)]])
            sims[j, pl.ds(16, 16)] = plsc.load_gather(
                grows[cur], [rowj, nrel[e, pl.ds(16, 16)]])
        base = GROUP * g
        if cur == 0:
            lax.fori_loop(base, base + GROUP,
                          lambda j, u: _topk_step(j, base, nrel, sims,
                                                  sela, iota16, neg_inf), 0)
        else:
            lax.fori_loop(base, base + GROUP,
                          lambda j, u: _topk_step(j, base, nrel, sims,
                                                  selb, iota16, neg_inf), 0)
        sel_cps[cur] = pltpu.async_copy(proj_hbm.at[sels[cur]],
                                        selrows[cur], sem_s[cur])
        if g > 0:
            sel_cps[1 - cur].wait()
            ctx_accum(selrows[1 - cur], g - 1)
    sel_cps[(NGRP - 1) % 2].wait()
    ctx_accum(selrows[(NGRP - 1) % 2], NGRP - 1)


def _topk_step(j, base, nrel, sims, sel_ref, iota16, neg_inf):
    jj = j - base
    nid0 = nrel[j, pl.ds(0, 16)]
    nid1 = nrel[j, pl.ds(16, 16)]
    s0 = sims[jj, pl.ds(0, 16)]
    s1 = sims[jj, pl.ds(16, 16)]
    for it in range(TOPK):
        m = jnp.max(jnp.maximum(s0, s1))
        eq0 = s0 == m
        eq1 = s1 == m
        c0 = plsc.all_reduce_population_count(eq0)
        f0 = plsc.all_reduce_ffs(eq0)
        f1 = plsc.all_reduce_ffs(eq1)
        use0 = c0 > 0
        cand0 = nid0.at[f0].get(mode="promise_in_bounds")
        cand1 = nid1.at[f1].get(mode="promise_in_bounds")
        sel_id = jnp.where(use0, cand0, cand1)
        plsc.store_scatter(sel_ref,
                           [jnp.full((16,), jj * TOPK + it, jnp.int32)],
                           sel_id, mask=iota16 == 0)
        s0 = jnp.where(use0 & (iota16 == f0), neg_inf, s0)
        s1 = jnp.where((~use0) & (iota16 == f1), neg_inf, s1)
    return 0


def _run_k2(ents, adj_rel_p, adj_nbr_p, g_mat, proj):
    mesh = plsc.VectorSubcoreMesh(core_axis_name="c", subcore_axis_name="s",
                                  num_cores=NC, num_subcores=NS)
    fn = pl.kernel(
        _sc_body,
        out_type=(jax.ShapeDtypeStruct((T, DIM), jnp.float32),
                  jax.ShapeDtypeStruct((T, DIM), jnp.float32)),
        mesh=mesh,
        compiler_params=pltpu.CompilerParams(needs_layout_passes=False),
        scratch_types=[
            pltpu.VMEM((128,), jnp.int32),
            pltpu.VMEM((16,), jnp.int32),
            pltpu.VMEM((16, 128), jnp.int32),
            pltpu.VMEM((16, 128), jnp.int32),
            pltpu.VMEM((E, 128), jnp.int32),
            pltpu.VMEM((E, DIM), jnp.float32),
            pltpu.VMEM((GROUP, NP), jnp.float32),
            pltpu.VMEM((GROUP, NP), jnp.float32),
            pltpu.VMEM((GROUP, DEG), jnp.float32),
            pltpu.VMEM((GROUP * TOPK,), jnp.int32),
            pltpu.VMEM((GROUP * TOPK,), jnp.int32),
            pltpu.VMEM((GROUP * TOPK, DIM), jnp.float32),
            pltpu.VMEM((GROUP * TOPK, DIM), jnp.float32),
            pltpu.VMEM((GROUP, DIM), jnp.float32),
            pltpu.SemaphoreType.DMA,
            pltpu.SemaphoreType.DMA,
            pltpu.SemaphoreType.DMA,
            pltpu.SemaphoreType.DMA,
            pltpu.SemaphoreType.DMA,
        ],
    )
    return fn(ents, adj_rel_p, adj_nbr_p, g_mat, proj)


def _layernorm(x, s, b):
    m = jnp.mean(x, axis=-1, keepdims=True)
    v = jnp.mean((x - m) ** 2, axis=-1, keepdims=True)
    return (x - m) / jnp.sqrt(v + EPS) * s + b


def _k3_body(states_ref, ctx_ref, wv_ref, bv_ref, wo_ref, bo_ref,
             ln1s_ref, ln1b_ref, w1_ref, b1_ref, w2_ref, b2_ref,
             ln2s_ref, ln2b_ref, out_ref):
    x = states_ref[...]
    ctx = ctx_ref[...]
    for l in range(LAYERS):
        v = jnp.dot(ctx, wv_ref[l], preferred_element_type=jnp.float32) \
            + bv_ref[l]
        attn = jnp.dot(v, wo_ref[l], preferred_element_type=jnp.float32) \
            + bo_ref[l]
        x = _layernorm(x + attn, ln1s_ref[l], ln1b_ref[l])
        h = jnp.dot(x, w1_ref[l], preferred_element_type=jnp.float32) \
            + b1_ref[l]
        h = 0.5 * h * (1.0 + lax.erf(h * jnp.float32(0.7071067811865476)))
        ff = jnp.dot(h, w2_ref[l], preferred_element_type=jnp.float32) \
            + b2_ref[l]
        x = _layernorm(x + ff, ln2s_ref[l], ln2b_ref[l])
    out_ref[...] = x


def _run_k3(states0, ctx, Wv, bv, Wo, bo, ln1_s, ln1_b, W1, b1, W2, b2,
            ln2_s, ln2_b):
    return pl.pallas_call(
        _k3_body,
        out_shape=jax.ShapeDtypeStruct((T, DIM), jnp.float32),
    )(states0, ctx, Wv, bv.reshape(LAYERS, 1, DIM), Wo,
      bo.reshape(LAYERS, 1, DIM), ln1_s.reshape(LAYERS, 1, DIM),
      ln1_b.reshape(LAYERS, 1, DIM), W1, b1.reshape(LAYERS, 1, 4 * DIM),
      W2, b2.reshape(LAYERS, 1, DIM), ln2_s.reshape(LAYERS, 1, DIM),
      ln2_b.reshape(LAYERS, 1, DIM))


def kernel(triple_ids, adj_rel, adj_nbr, rel_semantics, W_proj, b_proj,
           Wq, bq, Wk, bk, Wv, bv, Wo, bo, ln1_s, ln1_b,
           W1, b1, W2, b2, ln2_s, ln2_b):
    del Wq, bq, Wk, bk
    adj_rel = adj_rel.astype(jnp.int32)
    adj_nbr = adj_nbr.astype(jnp.int32)
    proj, g_mat, adj_rel_p, adj_nbr_p = _run_k1(
        rel_semantics, W_proj, b_proj, adj_rel, adj_nbr)

    head = triple_ids[:, 0].astype(jnp.int32)
    tail = triple_ids[:, 2].astype(jnp.int32)
    ents = jnp.zeros((128,), jnp.int32).at[0:B].set(head).at[B:2 * B].set(tail)

    states0, ctx = _run_k2(ents, adj_rel_p, adj_nbr_p, g_mat, proj)
    out = _run_k3(states0, ctx, Wv, bv, Wo, bo, ln1_s, ln1_b,
                  W1, b1, W2, b2, ln2_s, ln2_b)
    states = out.reshape(B, E, DIM)
    mask = jnp.ones((B, E), jnp.float32)
    return states, mask
